# baseline (speedup 1.0000x reference)
import functools

import jax
import jax.numpy as jnp
from jax import lax
from jax.experimental import pallas as pl
from jax.experimental.pallas import tpu as pltpu
from jax.experimental.pallas import tpu_sc as plsc

N = 10000
E = 320000
NP = 10240
NC = 2
NS = 16
EPT = E // NS
EGRP = EPT // 16
SLW = NP // NS
NSL = NP // (NC * NS)
GP = 128
SUBP = 1024

_MESH = plsc.VectorSubcoreMesh(core_axis_name="c", subcore_axis_name="s")
_SC_PARAMS = pltpu.CompilerParams(needs_layout_passes=False)


def _zero_f32(ref, n):
    z = jnp.zeros((16,), jnp.float32)

    @plsc.parallel_loop(0, n // 16, unroll=8)
    def _(i):
        ref[pl.ds(i * 16, 16)] = z


def _rsqrt16(x):
    xi = plsc.bitcast(x, jnp.int32)
    yi = jnp.full((16,), 0x5F3759DF, jnp.int32) - lax.shift_right_logical(
        xi, jnp.full((16,), 1, jnp.int32))
    y = plsc.bitcast(yi, jnp.float32)
    for _ in range(3):
        y = y * (1.5 - 0.5 * x * y * y)
    return y


@functools.partial(
    pl.kernel,
    out_type=jax.ShapeDtypeStruct((NC * 5 * GP,), jnp.float32),
    mesh=_MESH,
    compiler_params=_SC_PARAMS,
    scratch_types=[
        pltpu.VMEM((EPT,), jnp.int32),
        pltpu.VMEM((EPT,), jnp.int32),
        pltpu.VMEM((NP,), jnp.float32),
        pltpu.VMEM((NP,), jnp.float32),
        pltpu.VMEM((NP,), jnp.float32),
        pltpu.VMEM((NP,), jnp.float32),
        pltpu.VMEM((NP,), jnp.float32),
        pltpu.VMEM((NP,), jnp.float32),
        pltpu.VMEM((NP,), jnp.float32),
        pltpu.VMEM((SUBP,), jnp.int32),
        pltpu.VMEM((SLW,), jnp.float32),
        pltpu.VMEM((NSL,), jnp.float32),
        pltpu.VMEM((NSL,), jnp.float32),
        pltpu.VMEM((GP,), jnp.float32),
        pltpu.VMEM((GP,), jnp.float32),
        pltpu.VMEM((GP,), jnp.float32),
        pltpu.VMEM((GP,), jnp.float32),
        pltpu.VMEM((GP,), jnp.float32),
        pltpu.VMEM((NP,), jnp.int32),
        pltpu.VMEM((GP,), jnp.int32),
        pltpu.SemaphoreType.DMA,
        pltpu.VMEM_SHARED((NP,), jnp.float32),
        pltpu.VMEM_SHARED((NP,), jnp.float32),
        pltpu.VMEM_SHARED((NP,), jnp.float32),
        pltpu.VMEM_SHARED((NP,), jnp.float32),
        pltpu.VMEM_SHARED((GP,), jnp.float32),
        pltpu.VMEM_SHARED((GP,), jnp.float32),
        pltpu.VMEM_SHARED((GP,), jnp.float32),
        pltpu.VMEM_SHARED((GP,), jnp.float32),
        pltpu.VMEM_SHARED((GP,), jnp.float32),
    ],
)
def _sc_mega(src_hbm, dst_hbm, cur_hbm, n2sf_hbm, s2g_hbm, iota_hbm,
             out_hbm,
             src_v, dst_v, c0, c1, c2, a0, a1, a2, dv, s2g_v,
             sl0, d1_v, d2_v, p0, p1, p2, p3, p4,
             iota_v, gpidx, sem,
             sum0_sh, sum1_sh, sum2_sh, dinv_sh,
             q0_sh, q1_sh, q2_sh, q3_sh, q4_sh):
    sid = lax.axis_index("s")
    cid = lax.axis_index("c")
    be = sid * EPT
    cs = (c0, c1, c2)
    PSL = GP // NS

    cps = [
        pltpu.async_copy(src_hbm.at[pl.ds(be, EPT)], src_v, sem),
        pltpu.async_copy(dst_hbm.at[pl.ds(be, EPT)], dst_v, sem),
        pltpu.async_copy(cur_hbm.at[pl.ds(0 * NP, NP)], c0, sem),
        pltpu.async_copy(cur_hbm.at[pl.ds(1 * NP, NP)], c1, sem),
        pltpu.async_copy(cur_hbm.at[pl.ds(2 * NP, NP)], c2, sem),
        pltpu.async_copy(s2g_hbm, s2g_v, sem),
        pltpu.async_copy(iota_hbm, iota_v, sem),
    ]
    i16 = lax.iota(jnp.int32, 16)

    @plsc.parallel_loop(0, GP // 16, unroll=4)
    def _(g):
        gpidx[pl.ds(g * 16, 16)] = i16 + g * 16
    _zero_f32(sl0, SLW)
    pltpu.sync_copy(sl0, dinv_sh.at[pl.ds(sid * SLW, SLW)])
    for sh in (sum0_sh, sum1_sh, sum2_sh):
        pltpu.sync_copy(sl0, sh.at[pl.ds(sid * SLW, SLW)])
    for k, sh in enumerate((q0_sh, q1_sh, q2_sh, q3_sh, q4_sh)):
        pltpu.sync_copy(sl0.at[pl.ds(0, PSL)],
                        sh.at[pl.ds(sid * PSL, PSL)])
    _zero_f32(a0, NP)
    plsc.subcore_barrier()
    for cp in cps:
        cp.wait()

    one = jnp.ones((16,), jnp.float32)

    @plsc.parallel_loop(0, EGRP, unroll=8)
    def _(g):
        d = dst_v[pl.ds(g * 16, 16)]
        plsc.addupdate_scatter(a0, [d], one)

    pltpu.sync_copy(a0, dinv_sh.at[iota_v], add=True)
    plsc.subcore_barrier()

    pltpu.sync_copy(dinv_sh.at[pl.ds(sid * SLW, SLW)], sl0)

    @plsc.parallel_loop(0, SLW // 16, unroll=4)
    def _(g):
        sl = pl.ds(g * 16, 16)
        sl0[sl] = _rsqrt16(sl0[sl] + 1.0)

    pltpu.sync_copy(sl0, dinv_sh.at[pl.ds(sid * SLW, SLW)])
    plsc.subcore_barrier()
    pltpu.sync_copy(dinv_sh, dv)

    nxt = None
    for pidx in range(2):
        _zero_f32(a0, NP)
        _zero_f32(a1, NP)
        _zero_f32(a2, NP)
        if pidx == 1:
            _zero_f32(sl0, SLW)
            for sh in (sum0_sh, sum1_sh, sum2_sh):
                pltpu.sync_copy(sl0, sh.at[pl.ds(sid * SLW, SLW)])
            plsc.subcore_barrier()

        if pidx == 0:
            @plsc.parallel_loop(0, EGRP, unroll=8)
            def _(g):
                sl = pl.ds(g * 16, 16)
                s = src_v[sl]
                d = dst_v[sl]
                nw = plsc.load_gather(dv, [s]) * plsc.load_gather(dv, [d])
                plsc.addupdate_scatter(a0, [d], plsc.load_gather(c0, [s]) * nw)
                plsc.addupdate_scatter(a1, [d], plsc.load_gather(c1, [s]) * nw)
                plsc.addupdate_scatter(a2, [d], nw)
        else:
            @plsc.parallel_loop(0, EGRP, unroll=8)
            def _(g):
                sl = pl.ds(g * 16, 16)
                s = src_v[sl]
                d = dst_v[sl]
                nw = plsc.load_gather(dv, [s]) * plsc.load_gather(dv, [d])
                plsc.addupdate_scatter(a0, [d], plsc.load_gather(c0, [s]) * nw)
                plsc.addupdate_scatter(a1, [d], plsc.load_gather(c1, [s]) * nw)
                plsc.addupdate_scatter(a2, [d], plsc.load_gather(c2, [s]) * nw)

        accs = (a0, a1, a2)
        for c in range(3):
            cc = cs[c]
            acc = accs[c]

            @plsc.parallel_loop(0, SLW // 16, unroll=4)
            def _(g):
                nsl = pl.ds(sid * SLW + g * 16, 16)
                acc[nsl] = acc[nsl] + dv[nsl] * dv[nsl] * cc[nsl]

        pltpu.sync_copy(a0, sum0_sh.at[iota_v], add=True)
        pltpu.sync_copy(a1, sum1_sh.at[iota_v], add=True)
        pltpu.sync_copy(a2, sum2_sh.at[iota_v], add=True)
        plsc.subcore_barrier()

        dsave = d1_v if pidx == 0 else d2_v
        pltpu.sync_copy(
            sum2_sh.at[pl.ds(sid * SLW + cid * NSL, NSL)], dsave)
        pltpu.sync_copy(sum0_sh, c0)
        pltpu.sync_copy(sum1_sh, c1)
        if pidx == 0:
            pltpu.sync_copy(sum2_sh, c2)
            plsc.subcore_barrier()
        else:
            nxt = pltpu.async_copy(n2sf_hbm, a2, sem)

    for p in (p0, p1, p2, p3, p4):
        _zero_f32(p, GP)
    nxt.wait()

    @plsc.parallel_loop(0, EGRP // 2, unroll=8)
    def _(g):
        sl = pl.ds((cid * (EGRP // 2) + g) * 16, 16)
        s = src_v[sl]
        d = dst_v[sl]
        nw = plsc.load_gather(dv, [s]) * plsc.load_gather(dv, [d])
        ni = lax.convert_element_type(plsc.load_gather(a2, [d]), jnp.int32)
        gidx = plsc.load_gather(s2g_v, [ni])
        plsc.addupdate_scatter(p0, [gidx], plsc.load_gather(c0, [s]) * nw)
        plsc.addupdate_scatter(p1, [gidx], plsc.load_gather(c1, [s]) * nw)

    nbase = sid * SLW + cid * NSL

    @plsc.parallel_loop(0, NSL // 16, unroll=4)
    def _(g):
        sl = pl.ds(g * 16, 16)
        nsl = pl.ds(nbase + g * 16, 16)
        ni = lax.convert_element_type(a2[nsl], jnp.int32)
        gidx = plsc.load_gather(s2g_v, [ni])
        dd = dv[nsl] * dv[nsl]
        plsc.addupdate_scatter(p0, [gidx], dd * c0[nsl])
        plsc.addupdate_scatter(p1, [gidx], dd * c1[nsl])
        plsc.addupdate_scatter(p2, [gidx], d2_v[sl])
        plsc.addupdate_scatter(p3, [gidx], d1_v[sl])
        plsc.addupdate_scatter(p4, [gidx], one)

    pltpu.sync_copy(p0, q0_sh.at[gpidx], add=True)
    pltpu.sync_copy(p1, q1_sh.at[gpidx], add=True)
    pltpu.sync_copy(p2, q2_sh.at[gpidx], add=True)
    pltpu.sync_copy(p3, q3_sh.at[gpidx], add=True)
    pltpu.sync_copy(p4, q4_sh.at[gpidx], add=True)
    plsc.subcore_barrier()

    @pl.when(sid == 0)
    def _():
        for k, sh in enumerate((q0_sh, q1_sh, q2_sh, q3_sh, q4_sh)):
            pltpu.sync_copy(sh, out_hbm.at[pl.ds((cid * 5 + k) * GP, GP)])


def _tc_head_body(pool_ref, W1_ref, b1_ref, W2_ref, b2_ref, W3_ref, b3_ref,
                  l1w_ref, l1b_ref, l2w_ref, l2b_ref, out_ref):
    pooled = jnp.sum(pool_ref[...], axis=0)
    W2 = W2_ref[...]
    W3 = W3_ref[...]
    W123 = jnp.dot(jnp.dot(W1_ref[...], W2, preferred_element_type=jnp.float32),
                   W3, preferred_element_type=jnp.float32)
    r1 = jnp.dot(jnp.dot(b1_ref[...], W2, preferred_element_type=jnp.float32),
                 W3, preferred_element_type=jnp.float32)
    r2 = jnp.dot(b2_ref[...], W3, preferred_element_type=jnp.float32)
    dn = (((0,), (0,)), ((), ()))
    g = lax.dot_general(pooled[0:2], W123, dn,
                        preferred_element_type=jnp.float32)
    g = g + lax.dot_general(pooled[2:3], r1, dn,
                            preferred_element_type=jnp.float32)
    g = g + lax.dot_general(pooled[3:4], r2, dn,
                            preferred_element_type=jnp.float32)
    g = g + lax.dot_general(pooled[4:5], b3_ref[...], dn,
                            preferred_element_type=jnp.float32)
    t = jnp.dot(g, l1w_ref[...], preferred_element_type=jnp.float32)
    t = jnp.maximum(t + l1b_ref[...], 0.0)
    o = jnp.dot(t, l2w_ref[...], preferred_element_type=jnp.float32)
    o = o + l2b_ref[...]
    m = jnp.max(o, axis=1, keepdims=True)
    e = jnp.exp(o - m)
    out_ref[...] = (o - m) - jnp.log(jnp.sum(e, axis=1, keepdims=True))


def _tc_head(pool, W1, b1, W2, b2, W3, b3, l1w, l1b, l2w, l2b):
    return pl.pallas_call(
        _tc_head_body,
        out_shape=jax.ShapeDtypeStruct((GP, 2), jnp.float32),
    )(pool, W1, b1, W2, b2, W3, b3, l1w, l1b, l2w, l2b)


def kernel(x, edge_index, node_to_subgraph, subgraph_to_graph,
           W1, b1, W2, b2, W3, b3, lin1_W, lin1_b, lin2_W, lin2_b):
    src = edge_index[0]
    dst = edge_index[1]
    pad = NP - N
    cur0 = jnp.concatenate([
        jnp.pad(x[:, 0], (0, pad)),
        jnp.pad(x[:, 1], (0, pad)),
        jnp.pad(jnp.ones((N,), jnp.float32), (0, pad)),
    ])
    n2s_f = jnp.pad(node_to_subgraph, (0, pad),
                    constant_values=1000).astype(jnp.float32)
    s2g = jnp.pad(subgraph_to_graph, (0, SUBP - 1000),
                  constant_values=112).astype(jnp.int32)

    iota = jnp.arange(NP, dtype=jnp.int32)
    pool = _sc_mega(src, dst, cur0, n2s_f, s2g, iota)

    out = _tc_head(pool.reshape(NC, 5, GP), W1, b1.reshape(1, -1), W2,
                   b2.reshape(1, -1), W3, b3.reshape(1, -1), lin1_W,
                   lin1_b.reshape(1, -1), lin2_W, lin2_b.reshape(1, -1))
    return out[:100, :]


# speedup vs baseline: 1.1938x; 1.1938x over previous
import functools

import jax
import jax.numpy as jnp
from jax import lax
from jax.experimental import pallas as pl
from jax.experimental.pallas import tpu as pltpu
from jax.experimental.pallas import tpu_sc as plsc

N = 10000
E = 320000
NP = 10240
NC = 2
NS = 16
NW = NC * NS
EPT = E // NS
EGRP = EPT // 16
SLW = NP // NS
NSL = NP // NW
GP = 128
SUBP = 1024

_MESH = plsc.VectorSubcoreMesh(core_axis_name="c", subcore_axis_name="s")
_SC_PARAMS = pltpu.CompilerParams(needs_layout_passes=False)


def _zero_f32(ref, n):
    z = jnp.zeros((16,), jnp.float32)

    @plsc.parallel_loop(0, n // 16, unroll=8)
    def _(i):
        ref[pl.ds(i * 16, 16)] = z


def _rsqrt16(x):
    xi = plsc.bitcast(x, jnp.int32)
    yi = jnp.full((16,), 0x5F3759DF, jnp.int32) - lax.shift_right_logical(
        xi, jnp.full((16,), 1, jnp.int32))
    y = plsc.bitcast(yi, jnp.float32)
    for _ in range(3):
        y = y * (1.5 - 0.5 * x * y * y)
    return y


@functools.partial(
    pl.kernel,
    out_type=jax.ShapeDtypeStruct((NW * 5 * GP,), jnp.float32),
    mesh=_MESH,
    compiler_params=_SC_PARAMS,
    scratch_types=[
        pltpu.VMEM((EPT,), jnp.int32),
        pltpu.VMEM((EPT,), jnp.int32),
        pltpu.VMEM((NP,), jnp.float32),
        pltpu.VMEM((NP,), jnp.float32),
        pltpu.VMEM((NP,), jnp.float32),
        pltpu.VMEM((NP,), jnp.float32),
        pltpu.VMEM((NP,), jnp.float32),
        pltpu.VMEM((NP,), jnp.float32),
        pltpu.VMEM((NP,), jnp.float32),
        pltpu.VMEM((SUBP,), jnp.int32),
        pltpu.VMEM((SLW,), jnp.float32),
        pltpu.VMEM((SLW,), jnp.float32),
        pltpu.VMEM((SLW,), jnp.float32),
        pltpu.VMEM((NSL,), jnp.float32),
        pltpu.VMEM((NSL,), jnp.float32),
        pltpu.VMEM((GP,), jnp.float32),
        pltpu.VMEM((GP,), jnp.float32),
        pltpu.VMEM((GP,), jnp.float32),
        pltpu.VMEM((GP,), jnp.float32),
        pltpu.VMEM((GP,), jnp.float32),
        pltpu.SemaphoreType.DMA,
        pltpu.VMEM_SHARED((NS * NP,), jnp.float32),
        pltpu.VMEM_SHARED((3 * NP,), jnp.float32),
        pltpu.VMEM_SHARED((NP,), jnp.float32),
    ],
)
def _sc_mega(src_hbm, dst_hbm, xf_hbm, n2sf_hbm, s2g_hbm, out_hbm,
             src_v, dst_v, c0, c1, c2, a0, a1, a2, dv, s2g_v,
             sl0, sl1, sl2, d1_v, d2_v, p0, p1, p2, p3, p4, sem,
             red_sh, sum_sh, dinv_sh):
    sid = lax.axis_index("s")
    cid = lax.axis_index("c")
    w = sid * NC + cid
    be = sid * EPT
    cs = (c0, c1, c2)
    sls = (sl0, sl1, sl2)

    cps = [
        pltpu.async_copy(src_hbm.at[pl.ds(be, EPT)], src_v, sem),
        pltpu.async_copy(dst_hbm.at[pl.ds(be, EPT)], dst_v, sem),
        pltpu.async_copy(s2g_hbm, s2g_v, sem),
    ]
    cps.append(pltpu.async_copy(xf_hbm.at[pl.ds(0 * NP, NP)], c0, sem))
    cps.append(pltpu.async_copy(xf_hbm.at[pl.ds(1 * NP, NP)], c1, sem))
    _zero_f32(a0, NP)
    onev = jnp.ones((16,), jnp.float32)
    zv = jnp.zeros((16,), jnp.float32)

    @plsc.parallel_loop(0, N // 16, unroll=8)
    def _(i):
        c2[pl.ds(i * 16, 16)] = onev

    @plsc.parallel_loop(0, (NP - N) // 16, unroll=1)
    def _(i):
        c2[pl.ds(N + i * 16, 16)] = zv

    for cp in cps:
        cp.wait()

    one = jnp.ones((16,), jnp.float32)

    @plsc.parallel_loop(0, EGRP, unroll=8)
    def _(g):
        d = dst_v[pl.ds(g * 16, 16)]
        plsc.addupdate_scatter(a0, [d], one)

    pltpu.sync_copy(a0, red_sh.at[pl.ds(sid * NP, NP)])
    plsc.subcore_barrier()

    rcps = [pltpu.async_copy(
        red_sh.at[pl.ds(k * NP + sid * SLW, SLW)],
        a1.at[pl.ds(k * SLW, SLW)], sem) for k in range(NS)]
    for cp in rcps:
        cp.wait()

    @plsc.parallel_loop(0, SLW // 16, unroll=4)
    def _(g):
        sl = pl.ds(g * 16, 16)
        v = a1[pl.ds(0 * SLW + g * 16, 16)]
        for k in range(1, NS):
            v = v + a1[pl.ds(k * SLW + g * 16, 16)]
        sl0[sl] = _rsqrt16(v + 1.0)

    pltpu.sync_copy(sl0, dinv_sh.at[pl.ds(sid * SLW, SLW)])
    plsc.subcore_barrier()
    pltpu.sync_copy(dinv_sh, dv)

    for pidx in range(2):
        _zero_f32(a0, NP)
        _zero_f32(a1, NP)
        _zero_f32(a2, NP)

        if pidx == 0:
            @plsc.parallel_loop(0, EGRP, unroll=8)
            def _(g):
                sl = pl.ds(g * 16, 16)
                s = src_v[sl]
                d = dst_v[sl]
                nw = plsc.load_gather(dv, [s]) * plsc.load_gather(dv, [d])
                plsc.addupdate_scatter(a0, [d], plsc.load_gather(c0, [s]) * nw)
                plsc.addupdate_scatter(a1, [d], plsc.load_gather(c1, [s]) * nw)
                plsc.addupdate_scatter(a2, [d], nw)
        else:
            @plsc.parallel_loop(0, EGRP, unroll=8)
            def _(g):
                sl = pl.ds(g * 16, 16)
                s = src_v[sl]
                d = dst_v[sl]
                nw = plsc.load_gather(dv, [s]) * plsc.load_gather(dv, [d])
                plsc.addupdate_scatter(a0, [d], plsc.load_gather(c0, [s]) * nw)
                plsc.addupdate_scatter(a1, [d], plsc.load_gather(c1, [s]) * nw)
                plsc.addupdate_scatter(a2, [d], plsc.load_gather(c2, [s]) * nw)

        accs = (a0, a1, a2)
        for c in range(3):
            acc = accs[c]
            pltpu.sync_copy(acc, red_sh.at[pl.ds(sid * NP, NP)])
            plsc.subcore_barrier()
            ccps = [pltpu.async_copy(
                red_sh.at[pl.ds(k * NP + sid * SLW, SLW)],
                acc.at[pl.ds(k * SLW, SLW)], sem) for k in range(NS)]
            for cp in ccps:
                cp.wait()
            cc = cs[c]
            slc = sls[c]

            @plsc.parallel_loop(0, SLW // 16, unroll=4)
            def _(g):
                sl = pl.ds(g * 16, 16)
                v = acc[pl.ds(0 * SLW + g * 16, 16)]
                for k in range(1, NS):
                    v = v + acc[pl.ds(k * SLW + g * 16, 16)]
                dd = dv[pl.ds(sid * SLW + g * 16, 16)]
                slc[sl] = v + dd * dd * cc[pl.ds(sid * SLW + g * 16, 16)]

            pltpu.sync_copy(slc, sum_sh.at[pl.ds(c * NP + sid * SLW, SLW)])
            plsc.subcore_barrier()

        nxt = None
        if pidx == 1:
            nxt = pltpu.async_copy(n2sf_hbm, a2, sem)

        dsave = d1_v if pidx == 0 else d2_v

        @plsc.parallel_loop(0, NSL // 16, unroll=4)
        def _(g):
            dsave[pl.ds(g * 16, 16)] = sl2[pl.ds(cid * NSL + g * 16, 16)]

        pltpu.sync_copy(sum_sh.at[pl.ds(0 * NP, NP)], c0)
        pltpu.sync_copy(sum_sh.at[pl.ds(1 * NP, NP)], c1)
        if pidx == 0:
            pltpu.sync_copy(sum_sh.at[pl.ds(2 * NP, NP)], c2)
        else:
            nxt.wait()

    for p in (p0, p1, p2, p3, p4):
        _zero_f32(p, GP)

    @plsc.parallel_loop(0, EGRP // 2, unroll=8)
    def _(g):
        sl = pl.ds((cid * (EGRP // 2) + g) * 16, 16)
        s = src_v[sl]
        d = dst_v[sl]
        nw = plsc.load_gather(dv, [s]) * plsc.load_gather(dv, [d])
        ni = lax.convert_element_type(plsc.load_gather(a2, [d]), jnp.int32)
        gidx = plsc.load_gather(s2g_v, [ni])
        plsc.addupdate_scatter(p0, [gidx], plsc.load_gather(c0, [s]) * nw)
        plsc.addupdate_scatter(p1, [gidx], plsc.load_gather(c1, [s]) * nw)

    nbase = sid * SLW + cid * NSL

    @plsc.parallel_loop(0, NSL // 16, unroll=4)
    def _(g):
        sl = pl.ds(g * 16, 16)
        nsl = pl.ds(nbase + g * 16, 16)
        ni = lax.convert_element_type(a2[nsl], jnp.int32)
        gidx = plsc.load_gather(s2g_v, [ni])
        dd = dv[nsl] * dv[nsl]
        plsc.addupdate_scatter(p0, [gidx], dd * c0[nsl])
        plsc.addupdate_scatter(p1, [gidx], dd * c1[nsl])
        plsc.addupdate_scatter(p2, [gidx], d2_v[sl])
        plsc.addupdate_scatter(p3, [gidx], d1_v[sl])
        plsc.addupdate_scatter(p4, [gidx], one)

    pltpu.sync_copy(p0, out_hbm.at[pl.ds((w * 5 + 0) * GP, GP)])
    pltpu.sync_copy(p1, out_hbm.at[pl.ds((w * 5 + 1) * GP, GP)])
    pltpu.sync_copy(p2, out_hbm.at[pl.ds((w * 5 + 2) * GP, GP)])
    pltpu.sync_copy(p3, out_hbm.at[pl.ds((w * 5 + 3) * GP, GP)])
    pltpu.sync_copy(p4, out_hbm.at[pl.ds((w * 5 + 4) * GP, GP)])


def _tc_head_body(pool_ref, W1_ref, b1_ref, W2_ref, b2_ref, W3_ref, b3_ref,
                  l1w_ref, l1b_ref, l2w_ref, l2b_ref, out_ref):
    pooled = jnp.sum(pool_ref[...], axis=0)
    W2 = W2_ref[...]
    W3 = W3_ref[...]
    W123 = jnp.dot(jnp.dot(W1_ref[...], W2, preferred_element_type=jnp.float32),
                   W3, preferred_element_type=jnp.float32)
    r1 = jnp.dot(jnp.dot(b1_ref[...], W2, preferred_element_type=jnp.float32),
                 W3, preferred_element_type=jnp.float32)
    r2 = jnp.dot(b2_ref[...], W3, preferred_element_type=jnp.float32)
    dn = (((0,), (0,)), ((), ()))
    g = lax.dot_general(pooled[0:2], W123, dn,
                        preferred_element_type=jnp.float32)
    g = g + lax.dot_general(pooled[2:3], r1, dn,
                            preferred_element_type=jnp.float32)
    g = g + lax.dot_general(pooled[3:4], r2, dn,
                            preferred_element_type=jnp.float32)
    g = g + lax.dot_general(pooled[4:5], b3_ref[...], dn,
                            preferred_element_type=jnp.float32)
    t = jnp.dot(g, l1w_ref[...], preferred_element_type=jnp.float32)
    t = jnp.maximum(t + l1b_ref[...], 0.0)
    o = jnp.dot(t, l2w_ref[...], preferred_element_type=jnp.float32)
    o = o + l2b_ref[...]
    m = jnp.max(o, axis=1, keepdims=True)
    e = jnp.exp(o - m)
    out_ref[...] = (o - m) - jnp.log(jnp.sum(e, axis=1, keepdims=True))


def _tc_head(pool, W1, b1, W2, b2, W3, b3, l1w, l1b, l2w, l2b):
    return pl.pallas_call(
        _tc_head_body,
        out_shape=jax.ShapeDtypeStruct((GP, 2), jnp.float32),
    )(pool, W1, b1, W2, b2, W3, b3, l1w, l1b, l2w, l2b)


def kernel(x, edge_index, node_to_subgraph, subgraph_to_graph,
           W1, b1, W2, b2, W3, b3, lin1_W, lin1_b, lin2_W, lin2_b):
    src = edge_index[0]
    dst = edge_index[1]
    pad = NP - N
    x_flat = jnp.pad(x.T, ((0, 0), (0, pad))).reshape(-1)
    n2s_f = jnp.pad(node_to_subgraph, (0, pad),
                    constant_values=1000).astype(jnp.float32)
    s2g = jnp.pad(subgraph_to_graph, (0, SUBP - 1000),
                  constant_values=112).astype(jnp.int32)

    pool = _sc_mega(src, dst, x_flat, n2s_f, s2g)

    out = _tc_head(pool.reshape(NW, 5, GP), W1, b1.reshape(1, -1), W2,
                   b2.reshape(1, -1), W3, b3.reshape(1, -1), lin1_W,
                   lin1_b.reshape(1, -1), lin2_W, lin2_b.reshape(1, -1))
    return out[:100, :]
